# baseline (speedup 1.0000x reference)
import functools

import jax
import jax.numpy as jnp
from jax import lax
from jax.experimental import pallas as pl
from jax.experimental.pallas import tpu as pltpu
from jax.experimental.pallas import tpu_sc as plsc

NC = 2
NS = 16
NW = NC * NS
LANES = 16

CHUNK_BAGS = 2


def _make_relayout(V, D):
    D2 = 2 * D
    npanel = V // 128
    tail_w = V - 128 * npanel
    per_w = -(-(npanel + (1 if tail_w else 0)) // NW)
    panel_elems = 128 * D

    mesh = plsc.VectorSubcoreMesh(core_axis_name="c", subcore_axis_name="s")

    @functools.partial(
        pl.kernel,
        out_type=jax.ShapeDtypeStruct((V * D,), jnp.float32),
        mesh=mesh,
        scratch_types=[
            pltpu.VMEM((D, 128), jnp.float32),
            pltpu.VMEM((D, 64), jnp.float32),
            pltpu.VMEM((panel_elems,), jnp.float32),
        ],
        compiler_params=pltpu.CompilerParams(use_tc_tiling_on_sc=True, needs_layout_passes=False),
    )
    def relayout_kernel(wt_hbm, wl_hbm, in_v, in_tail_v, out_v):
        wid = lax.axis_index("s") * NC + lax.axis_index("c")
        iota = lax.iota(jnp.int32, LANES)

        def transpose_panel(src, xw):
            for x0 in range(0, xw, LANES):
                x = iota + x0
                rowpre = ((x >> 1) * D2) + ((x & 1) * D)
                for c in range(D):
                    v = src[c, pl.ds(x0, LANES)]
                    plsc.store_scatter(out_v, [rowpre + c], v)

        def body(g, _):
            j = wid + NW * g

            @pl.when(j < npanel)
            def _():
                pltpu.sync_copy(wt_hbm.at[:, pl.ds(128 * j, 128)], in_v)
                transpose_panel(in_v, 128)
                pltpu.sync_copy(
                    out_v, wl_hbm.at[pl.ds(panel_elems * j, panel_elems)]
                )

            if tail_w:
                @pl.when(j == npanel)
                def _():
                    pltpu.sync_copy(wt_hbm.at[:, pl.ds(128 * npanel, tail_w)], in_tail_v)
                    transpose_panel(in_tail_v, tail_w)
                    pltpu.sync_copy(
                        out_v.at[pl.ds(0, tail_w * D)],
                        wl_hbm.at[pl.ds(panel_elems * npanel, tail_w * D)],
                    )
            return 0

        lax.fori_loop(0, per_w, body, 0)

    return relayout_kernel


def _make_kernel(B, H, D):
    bags_per_w = B // NW
    idx_per_chunk = CHUNK_BAGS * H
    nchunk = bags_per_w // CHUNK_BAGS
    col_groups = D // LANES
    inv_h = 1.0 / H
    D2 = 2 * D
    off_groups = -(-idx_per_chunk // LANES)
    off_pad = off_groups * LANES

    mesh = plsc.VectorSubcoreMesh(core_axis_name="c", subcore_axis_name="s")

    nbuf = 2

    @functools.partial(
        pl.kernel,
        out_type=jax.ShapeDtypeStruct((B, D), jnp.float32),
        mesh=mesh,
        scratch_types=[
            pltpu.VMEM((nchunk, idx_per_chunk), jnp.int32),
            pltpu.VMEM((nchunk, off_pad), jnp.int32),
            pltpu.VMEM((nbuf, idx_per_chunk, D2), jnp.float32),
            pltpu.VMEM((bags_per_w, D), jnp.float32),
            [pltpu.SemaphoreType.DMA] * nbuf,
        ],
        compiler_params=pltpu.CompilerParams(use_tc_tiling_on_sc=False),
    )
    def bag_kernel(fr_hbm, off_hbm, wl_hbm, out_hbm, fr_v, off_v, rows_v, out_v, sems):
        wid = lax.axis_index("s") * NC + lax.axis_index("c")
        pltpu.sync_copy(fr_hbm.at[wid], fr_v)
        pltpu.sync_copy(off_hbm.at[wid], off_v)

        def start(j, b):
            pltpu.async_copy(wl_hbm.at[fr_v.at[j]], rows_v.at[b], sems[b])

        for b in range(nbuf):
            start(b, b)

        def chunk_body(j, b):
            pltpu.make_async_copy(
                wl_hbm.at[fr_v.at[j]], rows_v.at[b], sems[b]
            ).wait()
            offs = [off_v[j, pl.ds(k * LANES, LANES)] for k in range(off_groups)]
            for bag in range(CHUNK_BAGS):
                accs = None
                r0 = bag * H
                for r in range(H):
                    o = offs[(r0 + r) // LANES][(r0 + r) % LANES]
                    vals = [
                        rows_v[b, r0 + r, pl.ds(o + c * LANES, LANES)]
                        for c in range(col_groups)
                    ]
                    accs = vals if accs is None else [a + v for a, v in zip(accs, vals)]
                for c in range(col_groups):
                    out_v[j * CHUNK_BAGS + bag, pl.ds(c * LANES, LANES)] = accs[c] * inv_h

        def outer(g, _):
            j0 = g * nbuf
            for b in range(nbuf):
                j = j0 + b
                chunk_body(j, b)
                nxt = j + nbuf

                @pl.when(nxt < nchunk)
                def _():
                    start(nxt, b)

            return 0

        lax.fori_loop(0, nchunk // nbuf, outer, 0)
        for t in range(nchunk - nchunk % nbuf, nchunk):
            chunk_body(t, t % nbuf)
        pltpu.sync_copy(out_v, out_hbm.at[pl.ds(wid * bags_per_w, bags_per_w)])

    return bag_kernel


def kernel(text, weight):
    B, H = text.shape
    _, D = weight.shape
    t32 = text.astype(jnp.int32)
    nchunk = (B // NW) // CHUNK_BAGS
    ipc = CHUNK_BAGS * H
    pad = -(-ipc // LANES) * LANES - ipc
    fr = (t32 >> 1).reshape(NW, nchunk, ipc)
    off = ((t32 & 1) << 6).reshape(NW * nchunk, ipc)
    off = jnp.pad(off, ((0, 0), (0, pad))).reshape(NW, nchunk, ipc + pad)
    V = weight.shape[0]
    wl_flat = _make_relayout(V, D)(weight.T)
    wl = wl_flat.reshape(V // 2, 2 * D)
    return _make_kernel(B, H, D)(fr, off, wl)


# speedup vs baseline: 1.2271x; 1.2271x over previous
import functools

import jax
import jax.numpy as jnp
from jax import lax
from jax.experimental import pallas as pl
from jax.experimental.pallas import tpu as pltpu
from jax.experimental.pallas import tpu_sc as plsc

NC = 2
NS = 16
NW = NC * NS
LANES = 16

CHUNK_BAGS = 2


def _make_relayout(V, D):
    D2 = 2 * D
    npanel = V // 128
    tail_w = V - 128 * npanel
    per_w = -(-(npanel + (1 if tail_w else 0)) // NW)
    panel_elems = 128 * D

    mesh = plsc.VectorSubcoreMesh(core_axis_name="c", subcore_axis_name="s")

    @functools.partial(
        pl.kernel,
        out_type=jax.ShapeDtypeStruct((V * D,), jnp.float32),
        mesh=mesh,
        scratch_types=[
            [pltpu.VMEM((D, 128), jnp.float32)] * 2,
            pltpu.VMEM((D, 64), jnp.float32),
            [pltpu.VMEM((panel_elems,), jnp.float32)] * 2,
            [pltpu.SemaphoreType.DMA] * 2,
            [pltpu.SemaphoreType.DMA] * 2,
        ],
        compiler_params=pltpu.CompilerParams(use_tc_tiling_on_sc=True, needs_layout_passes=False),
    )
    def relayout_kernel(wt_hbm, wl_hbm, in_v, in_tail_v, out_v, isems, osems):
        wid = lax.axis_index("s") * NC + lax.axis_index("c")
        iota = lax.iota(jnp.int32, LANES)

        def transpose_panel(src, dst, xw):
            for x0 in range(0, xw, LANES):
                x = iota + x0
                rowpre = ((x >> 1) * D2) + ((x & 1) * D)
                for c in range(D):
                    v = src[c, pl.ds(x0, LANES)]
                    plsc.store_scatter(dst, [rowpre + c], v)

        def in_start(g, b):
            j = wid + NW * g

            @pl.when(j < npanel)
            def _():
                pltpu.async_copy(
                    wt_hbm.at[:, pl.ds(128 * j, 128)], in_v[b], isems[b]
                )

        def phase(g, b, prefetch=True):
            j = wid + NW * g

            @pl.when(j < npanel)
            def _():
                pltpu.make_async_copy(
                    wt_hbm.at[:, pl.ds(128 * j, 128)], in_v[b], isems[b]
                ).wait()

                @pl.when(g >= 2)
                def _():
                    pltpu.make_async_copy(
                        out_v[b],
                        wl_hbm.at[pl.ds(panel_elems * j, panel_elems)],
                        osems[b],
                    ).wait()

                transpose_panel(in_v[b], out_v[b], 128)
                pltpu.async_copy(
                    out_v[b],
                    wl_hbm.at[pl.ds(panel_elems * j, panel_elems)],
                    osems[b],
                )

            if prefetch:
                in_start(g + 2, b)

        in_start(0, 0)
        in_start(1, 1)

        def body(k, _):
            phase(2 * k, 0)
            phase(2 * k + 1, 1)
            return 0

        n_main = per_w - per_w % 2
        lax.fori_loop(0, n_main // 2, body, 0)
        for g in range(n_main, per_w):
            phase(g, g % 2, prefetch=False)

        for b in range(2):
            pltpu.make_async_copy(
                out_v[b],
                wl_hbm.at[pl.ds(0, panel_elems)],
                osems[b],
            ).wait()

        if tail_w:
            @pl.when(wid == npanel % NW)
            def _():
                pltpu.sync_copy(wt_hbm.at[:, pl.ds(128 * npanel, tail_w)], in_tail_v)
                transpose_panel(in_tail_v, out_v[0], tail_w)
                pltpu.sync_copy(
                    out_v[0].at[pl.ds(0, tail_w * D)],
                    wl_hbm.at[pl.ds(panel_elems * npanel, tail_w * D)],
                )

    return relayout_kernel


def _make_kernel(B, H, D):
    bags_per_w = B // NW
    idx_per_chunk = CHUNK_BAGS * H
    nchunk = bags_per_w // CHUNK_BAGS
    col_groups = D // LANES
    inv_h = 1.0 / H
    D2 = 2 * D
    off_groups = -(-idx_per_chunk // LANES)
    off_pad = off_groups * LANES

    mesh = plsc.VectorSubcoreMesh(core_axis_name="c", subcore_axis_name="s")

    nbuf = 2

    @functools.partial(
        pl.kernel,
        out_type=jax.ShapeDtypeStruct((B, D), jnp.float32),
        mesh=mesh,
        scratch_types=[
            pltpu.VMEM((nchunk, idx_per_chunk), jnp.int32),
            pltpu.VMEM((nchunk, off_pad), jnp.int32),
            pltpu.VMEM((nbuf, idx_per_chunk, D2), jnp.float32),
            pltpu.VMEM((bags_per_w, D), jnp.float32),
            [pltpu.SemaphoreType.DMA] * nbuf,
        ],
        compiler_params=pltpu.CompilerParams(use_tc_tiling_on_sc=False),
    )
    def bag_kernel(fr_hbm, off_hbm, wl_hbm, out_hbm, fr_v, off_v, rows_v, out_v, sems):
        wid = lax.axis_index("s") * NC + lax.axis_index("c")
        pltpu.sync_copy(fr_hbm.at[wid], fr_v)
        pltpu.sync_copy(off_hbm.at[wid], off_v)

        def start(j, b):
            pltpu.async_copy(wl_hbm.at[fr_v.at[j]], rows_v.at[b], sems[b])

        for b in range(nbuf):
            start(b, b)

        def chunk_body(j, b):
            pltpu.make_async_copy(
                wl_hbm.at[fr_v.at[j]], rows_v.at[b], sems[b]
            ).wait()
            offs = [off_v[j, pl.ds(k * LANES, LANES)] for k in range(off_groups)]
            for bag in range(CHUNK_BAGS):
                accs = None
                r0 = bag * H
                for r in range(H):
                    o = offs[(r0 + r) // LANES][(r0 + r) % LANES]
                    vals = [
                        rows_v[b, r0 + r, pl.ds(o + c * LANES, LANES)]
                        for c in range(col_groups)
                    ]
                    accs = vals if accs is None else [a + v for a, v in zip(accs, vals)]
                for c in range(col_groups):
                    out_v[j * CHUNK_BAGS + bag, pl.ds(c * LANES, LANES)] = accs[c] * inv_h

        def outer(g, _):
            j0 = g * nbuf
            for b in range(nbuf):
                j = j0 + b
                chunk_body(j, b)
                nxt = j + nbuf

                @pl.when(nxt < nchunk)
                def _():
                    start(nxt, b)

            return 0

        lax.fori_loop(0, nchunk // nbuf, outer, 0)
        for t in range(nchunk - nchunk % nbuf, nchunk):
            chunk_body(t, t % nbuf)
        pltpu.sync_copy(out_v, out_hbm.at[pl.ds(wid * bags_per_w, bags_per_w)])

    return bag_kernel


def kernel(text, weight):
    B, H = text.shape
    _, D = weight.shape
    t32 = text.astype(jnp.int32)
    nchunk = (B // NW) // CHUNK_BAGS
    ipc = CHUNK_BAGS * H
    pad = -(-ipc // LANES) * LANES - ipc
    fr = (t32 >> 1).reshape(NW, nchunk, ipc)
    off = ((t32 & 1) << 6).reshape(NW * nchunk, ipc)
    off = jnp.pad(off, ((0, 0), (0, pad))).reshape(NW, nchunk, ipc + pad)
    V = weight.shape[0]
    wl_flat = _make_relayout(V, D)(weight.T)
    wl = wl_flat.reshape(V // 2, 2 * D)
    return _make_kernel(B, H, D)(fr, off, wl)


# speedup vs baseline: 1.5252x; 1.2428x over previous
import functools

import jax
import jax.numpy as jnp
from jax import lax
from jax.experimental import pallas as pl
from jax.experimental.pallas import tpu as pltpu
from jax.experimental.pallas import tpu_sc as plsc

NC = 2
NS = 16
NW = NC * NS
LANES = 16

CHUNK_BAGS = 2


def _make_relayout(V, D):
    D2 = 2 * D
    npanel = V // 128
    tail_w = V - 128 * npanel
    per_w = -(-(npanel + (1 if tail_w else 0)) // NW)
    panel_elems = 128 * D

    mesh = plsc.VectorSubcoreMesh(core_axis_name="c", subcore_axis_name="s")

    @functools.partial(
        pl.kernel,
        out_type=jax.ShapeDtypeStruct((V * D,), jnp.float32),
        mesh=mesh,
        scratch_types=[
            [pltpu.VMEM((D, 128), jnp.float32)] * 2,
            pltpu.VMEM((D, 64), jnp.float32),
            [pltpu.VMEM((panel_elems,), jnp.float32)] * 2,
            [pltpu.SemaphoreType.DMA] * 2,
            [pltpu.SemaphoreType.DMA] * 2,
        ],
        compiler_params=pltpu.CompilerParams(use_tc_tiling_on_sc=True, needs_layout_passes=False),
    )
    def relayout_kernel(wt_hbm, wl_hbm, in_v, in_tail_v, out_v, isems, osems):
        wid = lax.axis_index("s") * NC + lax.axis_index("c")
        iota = lax.iota(jnp.int32, LANES)

        def transpose_panel(src, dst, xw):
            bat = 8
            for x0 in range(0, xw, LANES):
                rowpre = (iota + x0) * D
                for c0 in range(0, D, bat):
                    vs = [src[c0 + k, pl.ds(x0, LANES)] for k in range(bat)]
                    for k in range(bat):
                        plsc.store_scatter(dst, [rowpre + (c0 + k)], vs[k])

        def in_start(g, b):
            j = wid + NW * g

            @pl.when(j < npanel)
            def _():
                pltpu.async_copy(
                    wt_hbm.at[:, pl.ds(128 * j, 128)], in_v[b], isems[b]
                )

        def phase(g, b, prefetch=True):
            j = wid + NW * g

            @pl.when(j < npanel)
            def _():
                pltpu.make_async_copy(
                    wt_hbm.at[:, pl.ds(128 * j, 128)], in_v[b], isems[b]
                ).wait()

                @pl.when(g >= 2)
                def _():
                    pltpu.make_async_copy(
                        out_v[b],
                        wl_hbm.at[pl.ds(panel_elems * j, panel_elems)],
                        osems[b],
                    ).wait()

                transpose_panel(in_v[b], out_v[b], 128)
                pltpu.async_copy(
                    out_v[b],
                    wl_hbm.at[pl.ds(panel_elems * j, panel_elems)],
                    osems[b],
                )

            if prefetch:
                in_start(g + 2, b)

        in_start(0, 0)
        in_start(1, 1)

        def body(k, _):
            phase(2 * k, 0)
            phase(2 * k + 1, 1)
            return 0

        n_main = per_w - per_w % 2
        lax.fori_loop(0, n_main // 2, body, 0)
        for g in range(n_main, per_w):
            phase(g, g % 2, prefetch=False)

        for b in range(2):
            pltpu.make_async_copy(
                out_v[b],
                wl_hbm.at[pl.ds(0, panel_elems)],
                osems[b],
            ).wait()

        if tail_w:
            @pl.when(wid == npanel % NW)
            def _():
                pltpu.sync_copy(wt_hbm.at[:, pl.ds(128 * npanel, tail_w)], in_tail_v)
                transpose_panel(in_tail_v, out_v[0], tail_w)
                pltpu.sync_copy(
                    out_v[0].at[pl.ds(0, tail_w * D)],
                    wl_hbm.at[pl.ds(panel_elems * npanel, tail_w * D)],
                )

    return relayout_kernel


def _make_kernel(B, H, D):
    bags_per_w = B // NW
    idx_per_chunk = CHUNK_BAGS * H
    nchunk = bags_per_w // CHUNK_BAGS
    col_groups = D // LANES
    inv_h = 1.0 / H
    D2 = 2 * D
    off_groups = -(-idx_per_chunk // LANES)
    off_pad = off_groups * LANES

    mesh = plsc.VectorSubcoreMesh(core_axis_name="c", subcore_axis_name="s")

    nbuf = 2

    @functools.partial(
        pl.kernel,
        out_type=jax.ShapeDtypeStruct((B, D), jnp.float32),
        mesh=mesh,
        scratch_types=[
            pltpu.VMEM((nchunk, idx_per_chunk), jnp.int32),
            pltpu.VMEM((nchunk, off_pad), jnp.int32),
            pltpu.VMEM((nbuf, idx_per_chunk, D2), jnp.float32),
            pltpu.VMEM((bags_per_w, D), jnp.float32),
            [pltpu.SemaphoreType.DMA] * nbuf,
        ],
        compiler_params=pltpu.CompilerParams(use_tc_tiling_on_sc=False),
    )
    def bag_kernel(fr_hbm, off_hbm, wl_hbm, out_hbm, fr_v, off_v, rows_v, out_v, sems):
        wid = lax.axis_index("s") * NC + lax.axis_index("c")
        pltpu.sync_copy(fr_hbm.at[wid], fr_v)
        pltpu.sync_copy(off_hbm.at[wid], off_v)

        def start(j, b):
            pltpu.async_copy(wl_hbm.at[fr_v.at[j]], rows_v.at[b], sems[b])

        for b in range(nbuf):
            start(b, b)

        def chunk_body(j, b):
            pltpu.make_async_copy(
                wl_hbm.at[fr_v.at[j]], rows_v.at[b], sems[b]
            ).wait()
            offs = [off_v[j, pl.ds(k * LANES, LANES)] for k in range(off_groups)]
            for bag in range(CHUNK_BAGS):
                accs = None
                r0 = bag * H
                for r in range(H):
                    o = offs[(r0 + r) // LANES][(r0 + r) % LANES]
                    vals = [
                        rows_v[b, r0 + r, pl.ds(o + c * LANES, LANES)]
                        for c in range(col_groups)
                    ]
                    accs = vals if accs is None else [a + v for a, v in zip(accs, vals)]
                for c in range(col_groups):
                    out_v[j * CHUNK_BAGS + bag, pl.ds(c * LANES, LANES)] = accs[c] * inv_h

        def outer(g, _):
            j0 = g * nbuf
            for b in range(nbuf):
                j = j0 + b
                chunk_body(j, b)
                nxt = j + nbuf

                @pl.when(nxt < nchunk)
                def _():
                    start(nxt, b)

            return 0

        lax.fori_loop(0, nchunk // nbuf, outer, 0)
        for t in range(nchunk - nchunk % nbuf, nchunk):
            chunk_body(t, t % nbuf)
        pltpu.sync_copy(out_v, out_hbm.at[pl.ds(wid * bags_per_w, bags_per_w)])

    return bag_kernel


def kernel(text, weight):
    B, H = text.shape
    _, D = weight.shape
    t32 = text.astype(jnp.int32)
    nchunk = (B // NW) // CHUNK_BAGS
    ipc = CHUNK_BAGS * H
    pad = -(-ipc // LANES) * LANES - ipc
    fr = (t32 >> 1).reshape(NW, nchunk, ipc)
    off = ((t32 & 1) << 6).reshape(NW * nchunk, ipc)
    off = jnp.pad(off, ((0, 0), (0, pad))).reshape(NW, nchunk, ipc + pad)
    V = weight.shape[0]
    wl_flat = _make_relayout(V, D)(weight.T)
    wl = wl_flat.reshape(V // 2, 2 * D)
    return _make_kernel(B, H, D)(fr, off, wl)


# speedup vs baseline: 1.5289x; 1.0025x over previous
import functools

import jax
import jax.numpy as jnp
from jax import lax
from jax.experimental import pallas as pl
from jax.experimental.pallas import tpu as pltpu
from jax.experimental.pallas import tpu_sc as plsc

NC = 2
NS = 16
NW = NC * NS
LANES = 16

CHUNK_BAGS = 2


def _make_relayout(V, D):
    D2 = 2 * D
    npanel = V // 128
    tail_w = V - 128 * npanel
    per_w = -(-(npanel + (1 if tail_w else 0)) // NW)
    panel_elems = 128 * D

    mesh = plsc.VectorSubcoreMesh(core_axis_name="c", subcore_axis_name="s")

    @functools.partial(
        pl.kernel,
        out_type=jax.ShapeDtypeStruct((V * D,), jnp.float32),
        mesh=mesh,
        scratch_types=[
            [pltpu.VMEM((D, 128), jnp.float32)] * 4,
            pltpu.VMEM((D, 64), jnp.float32),
            [pltpu.VMEM((panel_elems,), jnp.float32)] * 4,
            [pltpu.SemaphoreType.DMA] * 4,
            [pltpu.SemaphoreType.DMA] * 4,
        ],
        compiler_params=pltpu.CompilerParams(use_tc_tiling_on_sc=True, needs_layout_passes=False),
    )
    def relayout_kernel(wt_hbm, wl_hbm, in_v, in_tail_v, out_v, isems, osems):
        wid = lax.axis_index("s") * NC + lax.axis_index("c")
        iota = lax.iota(jnp.int32, LANES)

        def transpose_panel(src, dst, xw):
            bat = 8
            for x0 in range(0, xw, LANES):
                rowpre = (iota + x0) * D
                for c0 in range(0, D, bat):
                    vs = [src[c0 + k, pl.ds(x0, LANES)] for k in range(bat)]
                    for k in range(bat):
                        plsc.store_scatter(dst, [rowpre + (c0 + k)], vs[k])

        def in_start(g, b):
            j = wid + NW * g

            @pl.when(j < npanel)
            def _():
                pltpu.async_copy(
                    wt_hbm.at[:, pl.ds(128 * j, 128)], in_v[b], isems[b]
                )

        def phase(g, b, prefetch=True):
            j = wid + NW * g

            @pl.when(j < npanel)
            def _():
                pltpu.make_async_copy(
                    wt_hbm.at[:, pl.ds(128 * j, 128)], in_v[b], isems[b]
                ).wait()

                @pl.when(g >= 4)
                def _():
                    pltpu.make_async_copy(
                        out_v[b],
                        wl_hbm.at[pl.ds(panel_elems * j, panel_elems)],
                        osems[b],
                    ).wait()

                transpose_panel(in_v[b], out_v[b], 128)
                pltpu.async_copy(
                    out_v[b],
                    wl_hbm.at[pl.ds(panel_elems * j, panel_elems)],
                    osems[b],
                )

            if prefetch:
                in_start(g + 4, b)

        for b in range(4):
            in_start(b, b)

        def body(k, _):
            for p in range(4):
                phase(4 * k + p, p)
            return 0

        n_main = per_w - per_w % 4
        lax.fori_loop(0, n_main // 4, body, 0)
        for g in range(n_main, per_w):
            phase(g, g % 4, prefetch=False)

        for b in range(4):
            pltpu.make_async_copy(
                out_v[b],
                wl_hbm.at[pl.ds(0, panel_elems)],
                osems[b],
            ).wait()

        if tail_w:
            @pl.when(wid == npanel % NW)
            def _():
                pltpu.sync_copy(wt_hbm.at[:, pl.ds(128 * npanel, tail_w)], in_tail_v)
                transpose_panel(in_tail_v, out_v[0], tail_w)
                pltpu.sync_copy(
                    out_v[0].at[pl.ds(0, tail_w * D)],
                    wl_hbm.at[pl.ds(panel_elems * npanel, tail_w * D)],
                )

    return relayout_kernel


def _make_kernel(B, H, D):
    bags_per_w = B // NW
    idx_per_chunk = CHUNK_BAGS * H
    nchunk = bags_per_w // CHUNK_BAGS
    col_groups = D // LANES
    inv_h = 1.0 / H
    D2 = 2 * D
    off_groups = -(-idx_per_chunk // LANES)
    off_pad = off_groups * LANES

    mesh = plsc.VectorSubcoreMesh(core_axis_name="c", subcore_axis_name="s")

    nbuf = 2

    @functools.partial(
        pl.kernel,
        out_type=jax.ShapeDtypeStruct((B, D), jnp.float32),
        mesh=mesh,
        scratch_types=[
            pltpu.VMEM((nchunk, idx_per_chunk), jnp.int32),
            pltpu.VMEM((nchunk, off_pad), jnp.int32),
            pltpu.VMEM((nbuf, idx_per_chunk, D2), jnp.float32),
            pltpu.VMEM((bags_per_w, D), jnp.float32),
            [pltpu.SemaphoreType.DMA] * nbuf,
        ],
        compiler_params=pltpu.CompilerParams(use_tc_tiling_on_sc=False),
    )
    def bag_kernel(fr_hbm, off_hbm, wl_hbm, out_hbm, fr_v, off_v, rows_v, out_v, sems):
        wid = lax.axis_index("s") * NC + lax.axis_index("c")
        pltpu.sync_copy(fr_hbm.at[wid], fr_v)
        pltpu.sync_copy(off_hbm.at[wid], off_v)

        def start(j, b):
            pltpu.async_copy(wl_hbm.at[fr_v.at[j]], rows_v.at[b], sems[b])

        for b in range(nbuf):
            start(b, b)

        def chunk_body(j, b):
            pltpu.make_async_copy(
                wl_hbm.at[fr_v.at[j]], rows_v.at[b], sems[b]
            ).wait()
            offs = [off_v[j, pl.ds(k * LANES, LANES)] for k in range(off_groups)]
            for bag in range(CHUNK_BAGS):
                accs = None
                r0 = bag * H
                for r in range(H):
                    o = offs[(r0 + r) // LANES][(r0 + r) % LANES]
                    vals = [
                        rows_v[b, r0 + r, pl.ds(o + c * LANES, LANES)]
                        for c in range(col_groups)
                    ]
                    accs = vals if accs is None else [a + v for a, v in zip(accs, vals)]
                for c in range(col_groups):
                    out_v[j * CHUNK_BAGS + bag, pl.ds(c * LANES, LANES)] = accs[c] * inv_h

        def outer(g, _):
            j0 = g * nbuf
            for b in range(nbuf):
                j = j0 + b
                chunk_body(j, b)
                nxt = j + nbuf

                @pl.when(nxt < nchunk)
                def _():
                    start(nxt, b)

            return 0

        lax.fori_loop(0, nchunk // nbuf, outer, 0)
        for t in range(nchunk - nchunk % nbuf, nchunk):
            chunk_body(t, t % nbuf)
        pltpu.sync_copy(out_v, out_hbm.at[pl.ds(wid * bags_per_w, bags_per_w)])

    return bag_kernel


def kernel(text, weight):
    B, H = text.shape
    _, D = weight.shape
    t32 = text.astype(jnp.int32)
    nchunk = (B // NW) // CHUNK_BAGS
    ipc = CHUNK_BAGS * H
    pad = -(-ipc // LANES) * LANES - ipc
    fr = (t32 >> 1).reshape(NW, nchunk, ipc)
    off = ((t32 & 1) << 6).reshape(NW * nchunk, ipc)
    off = jnp.pad(off, ((0, 0), (0, pad))).reshape(NW, nchunk, ipc + pad)
    V = weight.shape[0]
    wl_flat = _make_relayout(V, D)(weight.T)
    wl = wl_flat.reshape(V // 2, 2 * D)
    return _make_kernel(B, H, D)(fr, off, wl)


# speedup vs baseline: 1.6241x; 1.0622x over previous
import functools

import jax
import jax.numpy as jnp
from jax import lax
from jax.experimental import pallas as pl
from jax.experimental.pallas import tpu as pltpu
from jax.experimental.pallas import tpu_sc as plsc

NC = 2
NS = 16
NW = NC * NS
LANES = 16

CHUNK_BAGS = 2


def _make_relayout(V, D):
    D2 = 2 * D
    npanel = V // 128
    tail_w = V - 128 * npanel
    per_w = -(-(npanel + (1 if tail_w else 0)) // NW)
    panel_elems = 128 * D

    mesh = plsc.VectorSubcoreMesh(core_axis_name="c", subcore_axis_name="s")

    @functools.partial(
        pl.kernel,
        out_type=jax.ShapeDtypeStruct((V * D,), jnp.float32),
        mesh=mesh,
        scratch_types=[
            [pltpu.VMEM((D, 128), jnp.float32)] * 4,
            pltpu.VMEM((D, 64), jnp.float32),
            [pltpu.VMEM((panel_elems,), jnp.float32)] * 4,
            [pltpu.SemaphoreType.DMA] * 4,
            [pltpu.SemaphoreType.DMA] * 4,
        ],
        compiler_params=pltpu.CompilerParams(use_tc_tiling_on_sc=True, needs_layout_passes=False),
    )
    def relayout_kernel(wt_hbm, wl_hbm, in_v, in_tail_v, out_v, isems, osems):
        wid = lax.axis_index("s") * NC + lax.axis_index("c")
        iota = lax.iota(jnp.int32, LANES)

        def transpose_panel(src, dst, xw):
            for x0 in range(0, xw, LANES):
                rowpre = (iota + x0) * D

                @plsc.parallel_loop(0, D, 1, unroll=8)
                def _(c):
                    v = src[c, pl.ds(x0, LANES)]
                    plsc.store_scatter(dst, [rowpre + c], v)

        def in_start(g, b):
            j = wid + NW * g

            @pl.when(j < npanel)
            def _():
                pltpu.async_copy(
                    wt_hbm.at[:, pl.ds(128 * j, 128)], in_v[b], isems[b]
                )

        def phase(g, b, prefetch=True):
            j = wid + NW * g

            @pl.when(j < npanel)
            def _():
                pltpu.make_async_copy(
                    wt_hbm.at[:, pl.ds(128 * j, 128)], in_v[b], isems[b]
                ).wait()

                @pl.when(g >= 4)
                def _():
                    pltpu.make_async_copy(
                        out_v[b],
                        wl_hbm.at[pl.ds(panel_elems * j, panel_elems)],
                        osems[b],
                    ).wait()

                transpose_panel(in_v[b], out_v[b], 128)
                pltpu.async_copy(
                    out_v[b],
                    wl_hbm.at[pl.ds(panel_elems * j, panel_elems)],
                    osems[b],
                )

            if prefetch:
                in_start(g + 4, b)

        for b in range(4):
            in_start(b, b)

        def body(k, _):
            for p in range(4):
                phase(4 * k + p, p)
            return 0

        n_main = per_w - per_w % 4
        lax.fori_loop(0, n_main // 4, body, 0)
        for g in range(n_main, per_w):
            phase(g, g % 4, prefetch=False)

        for b in range(4):
            pltpu.make_async_copy(
                out_v[b],
                wl_hbm.at[pl.ds(0, panel_elems)],
                osems[b],
            ).wait()

        if tail_w:
            @pl.when(wid == npanel % NW)
            def _():
                pltpu.sync_copy(wt_hbm.at[:, pl.ds(128 * npanel, tail_w)], in_tail_v)
                transpose_panel(in_tail_v, out_v[0], tail_w)
                pltpu.sync_copy(
                    out_v[0].at[pl.ds(0, tail_w * D)],
                    wl_hbm.at[pl.ds(panel_elems * npanel, tail_w * D)],
                )

    return relayout_kernel


def _make_kernel(B, H, D):
    bags_per_w = B // NW
    idx_per_chunk = CHUNK_BAGS * H
    nchunk = bags_per_w // CHUNK_BAGS
    col_groups = D // LANES
    inv_h = 1.0 / H
    D2 = 2 * D
    off_groups = -(-idx_per_chunk // LANES)
    off_pad = off_groups * LANES

    mesh = plsc.VectorSubcoreMesh(core_axis_name="c", subcore_axis_name="s")

    nbuf = 2

    @functools.partial(
        pl.kernel,
        out_type=jax.ShapeDtypeStruct((B, D), jnp.float32),
        mesh=mesh,
        scratch_types=[
            pltpu.VMEM((nchunk, idx_per_chunk), jnp.int32),
            pltpu.VMEM((nchunk, off_pad), jnp.int32),
            pltpu.VMEM((nbuf, idx_per_chunk, D2), jnp.float32),
            pltpu.VMEM((bags_per_w, D), jnp.float32),
            [pltpu.SemaphoreType.DMA] * nbuf,
        ],
        compiler_params=pltpu.CompilerParams(use_tc_tiling_on_sc=False),
    )
    def bag_kernel(fr_hbm, off_hbm, wl_hbm, out_hbm, fr_v, off_v, rows_v, out_v, sems):
        wid = lax.axis_index("s") * NC + lax.axis_index("c")
        pltpu.sync_copy(fr_hbm.at[wid], fr_v)
        pltpu.sync_copy(off_hbm.at[wid], off_v)

        def start(j, b):
            pltpu.async_copy(wl_hbm.at[fr_v.at[j]], rows_v.at[b], sems[b])

        for b in range(nbuf):
            start(b, b)

        def chunk_body(j, b):
            pltpu.make_async_copy(
                wl_hbm.at[fr_v.at[j]], rows_v.at[b], sems[b]
            ).wait()
            offs = [off_v[j, pl.ds(k * LANES, LANES)] for k in range(off_groups)]
            for bag in range(CHUNK_BAGS):
                accs = None
                r0 = bag * H
                for r in range(H):
                    o = offs[(r0 + r) // LANES][(r0 + r) % LANES]
                    vals = [
                        rows_v[b, r0 + r, pl.ds(o + c * LANES, LANES)]
                        for c in range(col_groups)
                    ]
                    accs = vals if accs is None else [a + v for a, v in zip(accs, vals)]
                for c in range(col_groups):
                    out_v[j * CHUNK_BAGS + bag, pl.ds(c * LANES, LANES)] = accs[c] * inv_h

        def outer(g, _):
            j0 = g * nbuf
            for b in range(nbuf):
                j = j0 + b
                chunk_body(j, b)
                nxt = j + nbuf

                @pl.when(nxt < nchunk)
                def _():
                    start(nxt, b)

            return 0

        lax.fori_loop(0, nchunk // nbuf, outer, 0)
        for t in range(nchunk - nchunk % nbuf, nchunk):
            chunk_body(t, t % nbuf)
        pltpu.sync_copy(out_v, out_hbm.at[pl.ds(wid * bags_per_w, bags_per_w)])

    return bag_kernel


def kernel(text, weight):
    B, H = text.shape
    _, D = weight.shape
    t32 = text.astype(jnp.int32)
    nchunk = (B // NW) // CHUNK_BAGS
    ipc = CHUNK_BAGS * H
    pad = -(-ipc // LANES) * LANES - ipc
    fr = (t32 >> 1).reshape(NW, nchunk, ipc)
    off = ((t32 & 1) << 6).reshape(NW * nchunk, ipc)
    off = jnp.pad(off, ((0, 0), (0, pad))).reshape(NW, nchunk, ipc + pad)
    V = weight.shape[0]
    wl_flat = _make_relayout(V, D)(weight.T)
    wl = wl_flat.reshape(V // 2, 2 * D)
    return _make_kernel(B, H, D)(fr, off, wl)


# speedup vs baseline: 2.7537x; 1.6956x over previous
import functools

import jax
import jax.numpy as jnp
from jax import lax
from jax.experimental import pallas as pl
from jax.experimental.pallas import tpu as pltpu
from jax.experimental.pallas import tpu_sc as plsc

NC = 2
NS = 16
NW = NC * NS
LANES = 16

CHUNK_BAGS = 2


def _make_relayout(V, D):
    D2 = 2 * D
    npanel = V // 128
    tail_w = V - 128 * npanel
    per_w = -(-(npanel + (1 if tail_w else 0)) // NW)
    panel_elems = 128 * D

    mesh = plsc.VectorSubcoreMesh(core_axis_name="c", subcore_axis_name="s")

    @functools.partial(
        pl.kernel,
        out_type=jax.ShapeDtypeStruct((V * D,), jnp.float32),
        mesh=mesh,
        scratch_types=[
            [pltpu.VMEM((D, 128), jnp.float32)] * 2,
            pltpu.VMEM((D, 64), jnp.float32),
            [pltpu.VMEM((panel_elems,), jnp.float32)] * 2,
            [pltpu.SemaphoreType.DMA] * 2,
            [pltpu.SemaphoreType.DMA] * 2,
        ],
        compiler_params=pltpu.CompilerParams(use_tc_tiling_on_sc=True, needs_layout_passes=False),
    )
    def relayout_kernel(wt_hbm, wl_hbm, in_v, in_tail_v, out_v, isems, osems):
        wid = lax.axis_index("s") * NC + lax.axis_index("c")
        iota = lax.iota(jnp.int32, LANES)

        rowpat = [(iota + k) & (LANES - 1) for k in range(LANES)]
        dstpat = [iota * D + ((iota + k) & (LANES - 1)) for k in range(LANES)]

        def transpose_panel(src, dst, xw):
            @plsc.parallel_loop(0, xw, LANES)
            def _(x0):
                col_idx = iota + x0
                for c0 in range(0, D, LANES):
                    dbase = x0 * D + c0
                    for k in range(LANES):
                        v = plsc.load_gather(src, [rowpat[k] + c0, col_idx])
                        plsc.store_scatter(dst, [dstpat[k] + dbase], v)

        def in_start(g, b):
            j = wid + NW * g

            @pl.when(j < npanel)
            def _():
                pltpu.async_copy(
                    wt_hbm.at[:, pl.ds(128 * j, 128)], in_v[b], isems[b]
                )

        def phase(g, b, prefetch=True):
            j = wid + NW * g

            @pl.when(j < npanel)
            def _():
                pltpu.make_async_copy(
                    wt_hbm.at[:, pl.ds(128 * j, 128)], in_v[b], isems[b]
                ).wait()

                @pl.when(g >= 2)
                def _():
                    pltpu.make_async_copy(
                        out_v[b],
                        wl_hbm.at[pl.ds(panel_elems * j, panel_elems)],
                        osems[b],
                    ).wait()

                transpose_panel(in_v[b], out_v[b], 128)
                pltpu.async_copy(
                    out_v[b],
                    wl_hbm.at[pl.ds(panel_elems * j, panel_elems)],
                    osems[b],
                )

            if prefetch:
                in_start(g + 2, b)

        for b in range(2):
            in_start(b, b)

        def body(k, _):
            for p in range(2):
                phase(2 * k + p, p)
            return 0

        n_main = per_w - per_w % 2
        lax.fori_loop(0, n_main // 2, body, 0)
        for g in range(n_main, per_w):
            phase(g, g % 2, prefetch=False)

        for b in range(2):
            pltpu.make_async_copy(
                out_v[b],
                wl_hbm.at[pl.ds(0, panel_elems)],
                osems[b],
            ).wait()

        if tail_w:
            @pl.when(wid == npanel % NW)
            def _():
                pltpu.sync_copy(wt_hbm.at[:, pl.ds(128 * npanel, tail_w)], in_tail_v)
                transpose_panel(in_tail_v, out_v[0], tail_w)
                pltpu.sync_copy(
                    out_v[0].at[pl.ds(0, tail_w * D)],
                    wl_hbm.at[pl.ds(panel_elems * npanel, tail_w * D)],
                )

    return relayout_kernel


def _make_kernel(B, H, D):
    bags_per_w = B // NW
    idx_per_chunk = CHUNK_BAGS * H
    nchunk = bags_per_w // CHUNK_BAGS
    col_groups = D // LANES
    inv_h = 1.0 / H
    D2 = 2 * D
    off_groups = -(-idx_per_chunk // LANES)
    off_pad = off_groups * LANES

    mesh = plsc.VectorSubcoreMesh(core_axis_name="c", subcore_axis_name="s")

    nbuf = 2

    @functools.partial(
        pl.kernel,
        out_type=jax.ShapeDtypeStruct((B, D), jnp.float32),
        mesh=mesh,
        scratch_types=[
            pltpu.VMEM((nchunk, idx_per_chunk), jnp.int32),
            pltpu.VMEM((nchunk, off_pad), jnp.int32),
            pltpu.VMEM((nbuf, idx_per_chunk, D2), jnp.float32),
            pltpu.VMEM((bags_per_w, D), jnp.float32),
            [pltpu.SemaphoreType.DMA] * nbuf,
        ],
        compiler_params=pltpu.CompilerParams(use_tc_tiling_on_sc=False),
    )
    def bag_kernel(fr_hbm, off_hbm, wl_hbm, out_hbm, fr_v, off_v, rows_v, out_v, sems):
        wid = lax.axis_index("s") * NC + lax.axis_index("c")
        pltpu.sync_copy(fr_hbm.at[wid], fr_v)
        pltpu.sync_copy(off_hbm.at[wid], off_v)

        def start(j, b):
            pltpu.async_copy(wl_hbm.at[fr_v.at[j]], rows_v.at[b], sems[b])

        for b in range(nbuf):
            start(b, b)

        def chunk_body(j, b):
            pltpu.make_async_copy(
                wl_hbm.at[fr_v.at[j]], rows_v.at[b], sems[b]
            ).wait()
            offs = [off_v[j, pl.ds(k * LANES, LANES)] for k in range(off_groups)]
            for bag in range(CHUNK_BAGS):
                accs = None
                r0 = bag * H
                for r in range(H):
                    o = offs[(r0 + r) // LANES][(r0 + r) % LANES]
                    vals = [
                        rows_v[b, r0 + r, pl.ds(o + c * LANES, LANES)]
                        for c in range(col_groups)
                    ]
                    accs = vals if accs is None else [a + v for a, v in zip(accs, vals)]
                for c in range(col_groups):
                    out_v[j * CHUNK_BAGS + bag, pl.ds(c * LANES, LANES)] = accs[c] * inv_h

        def outer(g, _):
            j0 = g * nbuf
            for b in range(nbuf):
                j = j0 + b
                chunk_body(j, b)
                nxt = j + nbuf

                @pl.when(nxt < nchunk)
                def _():
                    start(nxt, b)

            return 0

        lax.fori_loop(0, nchunk // nbuf, outer, 0)
        for t in range(nchunk - nchunk % nbuf, nchunk):
            chunk_body(t, t % nbuf)
        pltpu.sync_copy(out_v, out_hbm.at[pl.ds(wid * bags_per_w, bags_per_w)])

    return bag_kernel


def kernel(text, weight):
    B, H = text.shape
    _, D = weight.shape
    t32 = text.astype(jnp.int32)
    nchunk = (B // NW) // CHUNK_BAGS
    ipc = CHUNK_BAGS * H
    pad = -(-ipc // LANES) * LANES - ipc
    fr = (t32 >> 1).reshape(NW, nchunk, ipc)
    off = ((t32 & 1) << 6).reshape(NW * nchunk, ipc)
    off = jnp.pad(off, ((0, 0), (0, pad))).reshape(NW, nchunk, ipc + pad)
    V = weight.shape[0]
    wl_flat = _make_relayout(V, D)(weight.T)
    wl = wl_flat.reshape(V // 2, 2 * D)
    return _make_kernel(B, H, D)(fr, off, wl)
